# Initial kernel scaffold; baseline (speedup 1.0000x reference)
import jax
import jax.numpy as jnp
from jax.experimental import pallas as pl


def kernel(static_emb, dynamic_emb_0, dynamic_emb_1, w1, attn_src1, attn_trg1, w2, attn_src2, attn_trg2, edge_index):
    raise NotImplementedError("write your pallas kernel here")


# speedup vs baseline: 25.5747x; 25.5747x over previous
import functools

import jax
import jax.numpy as jnp
from jax import lax
from jax.experimental import pallas as pl
from jax.experimental.pallas import tpu as pltpu
from jax.experimental.pallas import tpu_sc as plsc

_N = 10000
_E = 320000
_NC = 2
_NS = 16
_L = 16
_W1 = 144
_W2 = 16
_CHUNK = 80
_NPAD = 10240
_ROWS_PER_TILE = _NPAD // _NS


def _k1_body(d0, d1, st, w, bs, bt, table_ref, atrg_ref):
  bn = d0.shape[0]
  emb = jnp.concatenate([d0[...], d1[...], st[...]], axis=1)
  hp = jnp.dot(emb, w[...], preferred_element_type=jnp.float32)
  asrc = jnp.dot(hp, bs[...], preferred_element_type=jnp.float32)
  atrg = jnp.dot(hp, bt[...], preferred_element_type=jnp.float32)
  z = jnp.zeros((bn, _W1 - 130), jnp.float32)
  t0 = jnp.concatenate([hp[:, :128], asrc[:, 0:2], z], axis=1)
  t1 = jnp.concatenate([hp[:, 128:], asrc[:, 2:4], z], axis=1)
  table_ref[...] = jnp.stack([t0, t1], axis=0)
  atrg_ref[...] = jnp.concatenate(
      [atrg, jnp.zeros((bn, 12), jnp.float32)], axis=1)


def _k1(d0, d1, st, w2d, bsrc, btrg):
  bn = 1000
  grid = _N // bn
  return pl.pallas_call(
      _k1_body,
      grid=(grid,),
      in_specs=[
          pl.BlockSpec((bn, 32), lambda i: (i, 0)),
          pl.BlockSpec((bn, 32), lambda i: (i, 0)),
          pl.BlockSpec((bn, 64), lambda i: (i, 0)),
          pl.BlockSpec((128, 256), lambda i: (0, 0)),
          pl.BlockSpec((256, 4), lambda i: (0, 0)),
          pl.BlockSpec((256, 4), lambda i: (0, 0)),
      ],
      out_specs=[
          pl.BlockSpec((2, bn, _W1), lambda i: (0, i, 0)),
          pl.BlockSpec((bn, 16), lambda i: (i, 0)),
      ],
      out_shape=[
          jax.ShapeDtypeStruct((2, _N, _W1), jnp.float32),
          jax.ShapeDtypeStruct((_N, 16), jnp.float32),
      ],
  )(d0, d1, st, w2d, bsrc, btrg)


def _edge_kernel(width, nheads, per_tile_edges, split_cores,
                 table_hbm, atrg_hbm, src_hbm, trg_hbm, zeros_hbm, out_hbm,
                 src_v, trg_v, rows_v, atrg_v, staged_v, wbuf_v, acc, sem):
  cid = lax.axis_index("c")
  sid = lax.axis_index("s")
  chunks = per_tile_edges // _CHUNK
  hblk = (width - _L) // nheads if nheads > 1 else width
  pltpu.sync_copy(zeros_hbm, acc.at[pl.ds(sid * _ROWS_PER_TILE,
                                          _ROWS_PER_TILE)])
  plsc.subcore_barrier()

  iota = lax.iota(jnp.int32, _L)
  ngrp = _CHUNK // _L

  if split_cores:
    base = (cid * _NS + sid) * per_tile_edges
  else:
    base = sid * per_tile_edges

  def chunk_body(i, carry):
    off = base + i * _CHUNK
    pltpu.sync_copy(src_hbm.at[pl.ds(off, _CHUNK)], src_v)
    pltpu.sync_copy(trg_hbm.at[pl.ds(off, _CHUNK)], trg_v)
    if not split_cores:
      for g in range(ngrp):
        s = src_v[pl.ds(g * _L, _L)]
        src_v[pl.ds(g * _L, _L)] = s + cid * _N
    pltpu.async_copy(table_hbm.at[src_v], rows_v, sem).wait()
    pltpu.async_copy(atrg_hbm.at[trg_v], atrg_v, sem).wait()
    for g in range(ngrp):
      ridx = g * _L + iota
      for h in range(nheads):
        if nheads > 1:
          acol = jnp.full((_L,), 128 + h, jnp.int32)
          tcol = jnp.full((_L,), 2 * cid + h, jnp.int32)
        else:
          acol = jnp.full((_L,), 3, jnp.int32)
          tcol = jnp.full((_L,), 0, jnp.int32)
        a = plsc.load_gather(rows_v, [ridx, acol])
        b = plsc.load_gather(atrg_v, [ridx, tcol])
        e = a + b
        e = jnp.where(e >= 0.0, e, 0.2 * e)
        wbuf_v[pl.ds(h * _CHUNK + g * _L, _L)] = jnp.exp(e)
    for j in range(_CHUNK):
      wb = [plsc.load_gather(wbuf_v, [jnp.full((_L,), h * _CHUNK + j,
                                               jnp.int32)])
            for h in range(nheads)]
      if nheads > 1:
        for c in range(width // _L - 1):
          h = (c * _L) // hblk
          staged_v[j, pl.ds(c * _L, _L)] = (
              rows_v[j, pl.ds(c * _L, _L)] * wb[h])
        tail = jnp.where(iota == 0, wb[0],
                         jnp.where(iota == 1, wb[1], 0.0))
        staged_v[j, pl.ds(width - _L, _L)] = tail
      else:
        staged_v[j, pl.ds(0, _L)] = rows_v[j, pl.ds(0, _L)] * wb[0]
    pltpu.sync_copy(staged_v, acc.at[trg_v], add=True)
    return carry

  lax.fori_loop(0, chunks, chunk_body, 0)
  plsc.subcore_barrier()
  r0 = sid * _ROWS_PER_TILE
  pltpu.sync_copy(acc.at[pl.ds(r0, _ROWS_PER_TILE)],
                  out_hbm.at[cid, pl.ds(r0, _ROWS_PER_TILE)])


def _sc_edge_call(width, nheads, per_tile_edges, split_cores,
                  table, atrg_t, src, trg, zeros):
  mesh = plsc.VectorSubcoreMesh(core_axis_name="c", subcore_axis_name="s",
                                num_cores=_NC, num_subcores=_NS)
  body = functools.partial(_edge_kernel, width, nheads, per_tile_edges,
                           split_cores)
  fn = pl.kernel(
      body,
      out_type=jax.ShapeDtypeStruct((_NC, _NPAD, width), jnp.float32),
      mesh=mesh,
      compiler_params=pltpu.CompilerParams(use_tc_tiling_on_sc=False,
                                           needs_layout_passes=False),
      scratch_types=[
          pltpu.VMEM((_CHUNK,), jnp.int32),
          pltpu.VMEM((_CHUNK,), jnp.int32),
          pltpu.VMEM((_CHUNK, width), jnp.float32),
          pltpu.VMEM((_CHUNK, 16), jnp.float32),
          pltpu.VMEM((_CHUNK, width), jnp.float32),
          pltpu.VMEM((nheads * _CHUNK,), jnp.float32),
          pltpu.VMEM_SHARED((_NPAD, width), jnp.float32),
          pltpu.SemaphoreType.DMA,
      ],
  )
  return fn(table, atrg_t, src, trg, zeros)


def _k3_body(acc, w2r, as2, at2, table2_ref, atrg2_ref):
  a = acc[...]
  bn = a.shape[1]
  eps = 1e-16
  parts = []
  for c in range(2):
    num = a[c, :, 0:128]
    d0 = jnp.broadcast_to(a[c, :, 128:129], (bn, 64))
    d1 = jnp.broadcast_to(a[c, :, 129:130], (bn, 64))
    den = jnp.concatenate([d0, d1], axis=1)
    parts.append(num / (den + eps))
  out1 = jnp.concatenate(parts, axis=1)
  emb2 = jnp.where(out1 > 0.0, out1, jnp.exp(out1) - 1.0)
  hp2 = jnp.dot(emb2, w2r[...], preferred_element_type=jnp.float32)
  asrc2 = jnp.dot(hp2, as2[...], preferred_element_type=jnp.float32)
  atrg2 = jnp.dot(hp2, at2[...], preferred_element_type=jnp.float32)
  ones = jnp.ones((bn, 1), jnp.float32)
  table2_ref[...] = jnp.concatenate(
      [hp2, ones, asrc2, jnp.zeros((bn, 12), jnp.float32)], axis=1)
  atrg2_ref[...] = jnp.concatenate(
      [atrg2, jnp.zeros((bn, 15), jnp.float32)], axis=1)


def _k3(acc1, w2r, as2, at2):
  bn = 1000
  grid = _N // bn
  return pl.pallas_call(
      _k3_body,
      grid=(grid,),
      in_specs=[
          pl.BlockSpec((2, bn, _W1), lambda i: (0, i, 0)),
          pl.BlockSpec((256, 2), lambda i: (0, 0)),
          pl.BlockSpec((2, 1), lambda i: (0, 0)),
          pl.BlockSpec((2, 1), lambda i: (0, 0)),
      ],
      out_specs=[
          pl.BlockSpec((bn, 16), lambda i: (i, 0)),
          pl.BlockSpec((bn, 16), lambda i: (i, 0)),
      ],
      out_shape=[
          jax.ShapeDtypeStruct((_N, 16), jnp.float32),
          jax.ShapeDtypeStruct((_N, 16), jnp.float32),
      ],
  )(acc1, w2r, as2, at2)


def _k5_body(acc, out_ref):
  a = acc[...]
  num = a[0, :, 0:2] + a[1, :, 0:2]
  den = a[0, :, 2:3] + a[1, :, 2:3]
  x = num / (den + 1e-16)
  m = jnp.max(x, axis=1, keepdims=True)
  lse = m + jnp.log(jnp.sum(jnp.exp(x - m), axis=1, keepdims=True))
  out_ref[...] = x - lse


def _k5(acc2):
  return pl.pallas_call(
      _k5_body,
      out_shape=jax.ShapeDtypeStruct((_N, 2), jnp.float32),
  )(acc2)


@jax.jit
def kernel(static_emb, dynamic_emb_0, dynamic_emb_1, w1, attn_src1,
           attn_trg1, w2, attn_src2, attn_trg2, edge_index):
  w2d = jnp.transpose(w1, (1, 0, 2)).reshape(128, 256)
  bsrc = jax.scipy.linalg.block_diag(*[attn_src1[h] for h in range(4)])
  btrg = jax.scipy.linalg.block_diag(*[attn_trg1[h] for h in range(4)])
  w2r = w2[0]
  as2 = attn_src2[0]
  at2 = attn_trg2[0]

  table1, atrg1t = _k1(dynamic_emb_0, dynamic_emb_1, static_emb,
                       w2d, bsrc, btrg)
  table1 = table1.reshape(2 * _N, _W1)

  src = edge_index[0]
  trg = edge_index[1]
  z1 = jnp.zeros((_ROWS_PER_TILE, _W1), jnp.float32)
  acc1 = _sc_edge_call(_W1, 2, _E // _NS, False,
                       table1, atrg1t, src, trg, z1)[:, :_N]

  table2, atrg2t = _k3(acc1, w2r, as2, at2)

  z2 = jnp.zeros((_ROWS_PER_TILE, _W2), jnp.float32)
  acc2 = _sc_edge_call(_W2, 1, _E // (_NC * _NS), True,
                       table2, atrg2t, src, trg, z2)[:, :_N]

  return _k5(acc2)


# speedup vs baseline: 33.0048x; 1.2905x over previous
import functools

import jax
import jax.numpy as jnp
from jax import lax
from jax.experimental import pallas as pl
from jax.experimental.pallas import tpu as pltpu
from jax.experimental.pallas import tpu_sc as plsc

_N = 10000
_E = 320000
_NC = 2
_NS = 16
_L = 16
_W1 = 144
_W2 = 16
_NPAD = 10240
_ROWS_PER_TILE = _NPAD // _NS


def _k1_body(d0, d1, st, w, bs, bt, table_ref, atrg_ref):
  bn = d0.shape[0]
  emb = jnp.concatenate([d0[...], d1[...], st[...]], axis=1)
  hp = jnp.dot(emb, w[...], preferred_element_type=jnp.float32)
  asrc = jnp.dot(hp, bs[...], preferred_element_type=jnp.float32)
  atrg = jnp.dot(hp, bt[...], preferred_element_type=jnp.float32)
  z = jnp.zeros((bn, _W1 - 130), jnp.float32)
  t0 = jnp.concatenate([hp[:, :128], asrc[:, 0:2], z], axis=1)
  t1 = jnp.concatenate([hp[:, 128:], asrc[:, 2:4], z], axis=1)
  table_ref[...] = jnp.stack([t0, t1], axis=0)
  atrg_ref[...] = jnp.concatenate(
      [atrg, jnp.zeros((bn, 12), jnp.float32)], axis=1)


def _k1(d0, d1, st, w2d, bsrc, btrg):
  bn = 1000
  grid = _N // bn
  return pl.pallas_call(
      _k1_body,
      grid=(grid,),
      in_specs=[
          pl.BlockSpec((bn, 32), lambda i: (i, 0)),
          pl.BlockSpec((bn, 32), lambda i: (i, 0)),
          pl.BlockSpec((bn, 64), lambda i: (i, 0)),
          pl.BlockSpec((128, 256), lambda i: (0, 0)),
          pl.BlockSpec((256, 4), lambda i: (0, 0)),
          pl.BlockSpec((256, 4), lambda i: (0, 0)),
      ],
      out_specs=[
          pl.BlockSpec((2, bn, _W1), lambda i: (0, i, 0)),
          pl.BlockSpec((bn, 16), lambda i: (i, 0)),
      ],
      out_shape=[
          jax.ShapeDtypeStruct((2, _N, _W1), jnp.float32),
          jax.ShapeDtypeStruct((_N, 16), jnp.float32),
      ],
  )(d0, d1, st, w2d, bsrc, btrg)


def _edge_kernel(width, nheads, per_tile_edges, split_cores, chunk,
                 table_hbm, atrg_hbm, src_hbm, trg_hbm, zeros_hbm, out_hbm,
                 src_v, trg_v, rows_v, atrg_v, staged_v, wbuf_v, acc,
                 sem_g, sem_s):
  cid = lax.axis_index("c")
  sid = lax.axis_index("s")
  chunks = -(-per_tile_edges // chunk)
  chunks_pad = chunks + (chunks % 2)
  npairs = chunks_pad // 2
  hblk = (width - _L) // nheads if nheads > 1 else width
  pltpu.sync_copy(zeros_hbm, acc.at[pl.ds(sid * _ROWS_PER_TILE,
                                          _ROWS_PER_TILE)])
  plsc.subcore_barrier()

  iota = lax.iota(jnp.int32, _L)
  ngrp = chunk // _L

  if split_cores:
    base = (cid * _NS + sid) * per_tile_edges
  else:
    base = sid * per_tile_edges

  end = base + per_tile_edges

  def load_idx_fire_gather(k, b):
    off_raw = base + k * chunk
    off = jnp.minimum(off_raw, _E - chunk)
    pltpu.sync_copy(src_hbm.at[pl.ds(off, chunk)], src_v[b])
    pltpu.sync_copy(trg_hbm.at[pl.ds(off, chunk)], trg_v[b])
    for g in range(ngrp):
      if not split_cores:
        s = src_v[b][pl.ds(g * _L, _L)]
        src_v[b][pl.ds(g * _L, _L)] = s + cid * _N
      gidx = off + g * _L + iota
      valid = (gidx >= off_raw) & (gidx < end)
      t = trg_v[b][pl.ds(g * _L, _L)]
      trg_v[b][pl.ds(g * _L, _L)] = jnp.where(valid, t, _N + 64)
    pltpu.async_copy(table_hbm.at[src_v[b]], rows_v[b], sem_g[b])
    pltpu.async_copy(atrg_hbm.at[trg_v[b]], atrg_v[b], sem_g[b])

  def wait_gather(b):
    pltpu.make_async_copy(table_hbm.at[src_v[b]], rows_v[b], sem_g[b]).wait()
    pltpu.make_async_copy(atrg_hbm.at[trg_v[b]], atrg_v[b], sem_g[b]).wait()

  def wait_scatter(b):
    pltpu.make_async_copy(staged_v[b], acc.at[trg_v[b]], sem_s[b]).wait()

  def compute_scale(b):
    for g in range(ngrp):
      ridx = g * _L + iota
      for h in range(nheads):
        if nheads > 1:
          acol = jnp.full((_L,), 128 + h, jnp.int32)
          tcol = jnp.full((_L,), 2 * cid + h, jnp.int32)
        else:
          acol = jnp.full((_L,), 3, jnp.int32)
          tcol = jnp.full((_L,), 0, jnp.int32)
        a = plsc.load_gather(rows_v[b], [ridx, acol])
        t = plsc.load_gather(atrg_v[b], [ridx, tcol])
        e = a + t
        e = jnp.where(e >= 0.0, e, 0.2 * e)
        wbuf_v[pl.ds(h * chunk + g * _L, _L)] = jnp.exp(e)
    for j in range(chunk):
      wb = [plsc.load_gather(wbuf_v, [jnp.full((_L,), h * chunk + j,
                                               jnp.int32)])
            for h in range(nheads)]
      if nheads > 1:
        for c in range(width // _L - 1):
          h = (c * _L) // hblk
          staged_v[b][j, pl.ds(c * _L, _L)] = (
              rows_v[b][j, pl.ds(c * _L, _L)] * wb[h])
        tail = jnp.where(iota == 0, wb[0],
                         jnp.where(iota == 1, wb[1], 0.0))
        staged_v[b][j, pl.ds(width - _L, _L)] = tail
      else:
        staged_v[b][j, pl.ds(0, _L)] = rows_v[b][j, pl.ds(0, _L)] * wb[0]

  load_idx_fire_gather(jnp.int32(0), 0)

  def pair_body(p, carry):
    for b in (0, 1):
      k = 2 * p + b
      wait_gather(b)
      if b == 0:
        @pl.when(p > 0)
        def _():
          wait_scatter(1)
      else:
        wait_scatter(0)
      load_idx_fire_gather(k + 1, 1 - b)
      compute_scale(b)
      pltpu.async_copy(staged_v[b], acc.at[trg_v[b]], sem_s[b], add=True)
    return carry

  lax.fori_loop(0, npairs, pair_body, 0)
  wait_scatter(1)
  wait_gather(0)
  plsc.subcore_barrier()
  r0 = sid * _ROWS_PER_TILE
  pltpu.sync_copy(acc.at[pl.ds(r0, _ROWS_PER_TILE)],
                  out_hbm.at[cid, pl.ds(r0, _ROWS_PER_TILE)])


def _sc_edge_call(width, nheads, per_tile_edges, split_cores, chunk,
                  table, atrg_t, src, trg, zeros):
  mesh = plsc.VectorSubcoreMesh(core_axis_name="c", subcore_axis_name="s",
                                num_cores=_NC, num_subcores=_NS)
  body = functools.partial(_edge_kernel, width, nheads, per_tile_edges,
                           split_cores, chunk)
  fn = pl.kernel(
      body,
      out_type=jax.ShapeDtypeStruct((_NC, _NPAD, width), jnp.float32),
      mesh=mesh,
      compiler_params=pltpu.CompilerParams(use_tc_tiling_on_sc=False,
                                           needs_layout_passes=False),
      scratch_types=[
          [pltpu.VMEM((chunk,), jnp.int32) for _ in range(2)],
          [pltpu.VMEM((chunk,), jnp.int32) for _ in range(2)],
          [pltpu.VMEM((chunk, width), jnp.float32) for _ in range(2)],
          [pltpu.VMEM((chunk, 16), jnp.float32) for _ in range(2)],
          [pltpu.VMEM((chunk, width), jnp.float32) for _ in range(2)],
          pltpu.VMEM((nheads * chunk,), jnp.float32),
          pltpu.VMEM_SHARED((_NPAD, width), jnp.float32),
          [pltpu.SemaphoreType.DMA for _ in range(2)],
          [pltpu.SemaphoreType.DMA for _ in range(2)],
      ],
  )
  return fn(table, atrg_t, src, trg, zeros)


def _k3_body(acc, w2r, as2, at2, table2_ref, atrg2_ref):
  a = acc[...]
  bn = a.shape[1]
  eps = 1e-16
  parts = []
  for c in range(2):
    num = a[c, :, 0:128]
    d0 = jnp.broadcast_to(a[c, :, 128:129], (bn, 64))
    d1 = jnp.broadcast_to(a[c, :, 129:130], (bn, 64))
    den = jnp.concatenate([d0, d1], axis=1)
    parts.append(num / (den + eps))
  out1 = jnp.concatenate(parts, axis=1)
  emb2 = jnp.where(out1 > 0.0, out1, jnp.exp(out1) - 1.0)
  hp2 = jnp.dot(emb2, w2r[...], preferred_element_type=jnp.float32)
  asrc2 = jnp.dot(hp2, as2[...], preferred_element_type=jnp.float32)
  atrg2 = jnp.dot(hp2, at2[...], preferred_element_type=jnp.float32)
  ones = jnp.ones((bn, 1), jnp.float32)
  table2_ref[...] = jnp.concatenate(
      [hp2, ones, asrc2, jnp.zeros((bn, 12), jnp.float32)], axis=1)
  atrg2_ref[...] = jnp.concatenate(
      [atrg2, jnp.zeros((bn, 15), jnp.float32)], axis=1)


def _k3(acc1, w2r, as2, at2):
  bn = 1000
  grid = _N // bn
  return pl.pallas_call(
      _k3_body,
      grid=(grid,),
      in_specs=[
          pl.BlockSpec((2, bn, _W1), lambda i: (0, i, 0)),
          pl.BlockSpec((256, 2), lambda i: (0, 0)),
          pl.BlockSpec((2, 1), lambda i: (0, 0)),
          pl.BlockSpec((2, 1), lambda i: (0, 0)),
      ],
      out_specs=[
          pl.BlockSpec((bn, 16), lambda i: (i, 0)),
          pl.BlockSpec((bn, 16), lambda i: (i, 0)),
      ],
      out_shape=[
          jax.ShapeDtypeStruct((_N, 16), jnp.float32),
          jax.ShapeDtypeStruct((_N, 16), jnp.float32),
      ],
  )(acc1, w2r, as2, at2)


def _k5_body(acc, out_ref):
  a = acc[...]
  num = a[0, :, 0:2] + a[1, :, 0:2]
  den = a[0, :, 2:3] + a[1, :, 2:3]
  x = num / (den + 1e-16)
  m = jnp.max(x, axis=1, keepdims=True)
  lse = m + jnp.log(jnp.sum(jnp.exp(x - m), axis=1, keepdims=True))
  out_ref[...] = x - lse


def _k5(acc2):
  return pl.pallas_call(
      _k5_body,
      out_shape=jax.ShapeDtypeStruct((_N, 2), jnp.float32),
  )(acc2)


@jax.jit
def kernel(static_emb, dynamic_emb_0, dynamic_emb_1, w1, attn_src1,
           attn_trg1, w2, attn_src2, attn_trg2, edge_index):
  w2d = jnp.transpose(w1, (1, 0, 2)).reshape(128, 256)
  bsrc = jax.scipy.linalg.block_diag(*[attn_src1[h] for h in range(4)])
  btrg = jax.scipy.linalg.block_diag(*[attn_trg1[h] for h in range(4)])
  w2r = w2[0]
  as2 = attn_src2[0]
  at2 = attn_trg2[0]

  table1, atrg1t = _k1(dynamic_emb_0, dynamic_emb_1, static_emb,
                       w2d, bsrc, btrg)
  table1 = table1.reshape(2 * _N, _W1)

  src = edge_index[0]
  trg = edge_index[1]
  z1 = jnp.zeros((_ROWS_PER_TILE, _W1), jnp.float32)
  acc1 = _sc_edge_call(_W1, 2, _E // _NS, False, 48,
                       table1, atrg1t, src, trg, z1)[:, :_N]

  table2, atrg2t = _k3(acc1, w2r, as2, at2)

  z2 = jnp.zeros((_ROWS_PER_TILE, _W2), jnp.float32)
  acc2 = _sc_edge_call(_W2, 1, _E // (_NC * _NS), True, 160,
                       table2, atrg2t, src, trg, z2)[:, :_N]

  return _k5(acc2)


# speedup vs baseline: 43.0686x; 1.3049x over previous
import functools

import jax
import jax.numpy as jnp
from jax import lax
from jax.experimental import pallas as pl
from jax.experimental.pallas import tpu as pltpu
from jax.experimental.pallas import tpu_sc as plsc

_N = 10000
_E = 320000
_NC = 2
_NS = 16
_L = 16
_W1 = 144
_W2 = 16
_NPAD = 10240
_ROWS_PER_TILE = _NPAD // _NS


def _k1_body(d0, d1, st, w, bs, bt, table_ref, atrg_ref):
  bn = d0.shape[0]
  emb = jnp.concatenate([d0[...], d1[...], st[...]], axis=1)
  hp = jnp.dot(emb, w[...], preferred_element_type=jnp.float32)
  asrc = jnp.dot(hp, bs[...], preferred_element_type=jnp.float32)
  atrg = jnp.dot(hp, bt[...], preferred_element_type=jnp.float32)
  z = jnp.zeros((bn, _W1 - 130), jnp.float32)
  t0 = jnp.concatenate([hp[:, :128], asrc[:, 0:2], z], axis=1)
  t1 = jnp.concatenate([hp[:, 128:], asrc[:, 2:4], z], axis=1)
  table_ref[...] = jnp.stack([t0, t1], axis=0)
  atrg_ref[...] = jnp.concatenate(
      [atrg, jnp.zeros((bn, 12), jnp.float32)], axis=1)


def _k1(d0, d1, st, w2d, bsrc, btrg):
  bn = 1000
  grid = _N // bn
  return pl.pallas_call(
      _k1_body,
      grid=(grid,),
      in_specs=[
          pl.BlockSpec((bn, 32), lambda i: (i, 0)),
          pl.BlockSpec((bn, 32), lambda i: (i, 0)),
          pl.BlockSpec((bn, 64), lambda i: (i, 0)),
          pl.BlockSpec((128, 256), lambda i: (0, 0)),
          pl.BlockSpec((256, 4), lambda i: (0, 0)),
          pl.BlockSpec((256, 4), lambda i: (0, 0)),
      ],
      out_specs=[
          pl.BlockSpec((2, bn, _W1), lambda i: (0, i, 0)),
          pl.BlockSpec((bn, 16), lambda i: (i, 0)),
      ],
      out_shape=[
          jax.ShapeDtypeStruct((2, _N, _W1), jnp.float32),
          jax.ShapeDtypeStruct((_N, 16), jnp.float32),
      ],
  )(d0, d1, st, w2d, bsrc, btrg)


def _edge_kernel(width, nheads, per_tile_edges, split_cores, chunk,
                 table_hbm, atrg_hbm, ei_hbm, zeros_hbm, out_hbm,
                 idxraw_v, src_v, trg_v, strg_v, rows_v, atrg_v, staged_v,
                 wbuf_v, acc, sem_i, sem_g, sem_s):
  cid = lax.axis_index("c")
  sid = lax.axis_index("s")
  chunks = -(-per_tile_edges // chunk)
  chunks_pad = chunks + (chunks % 2)
  npairs = chunks_pad // 2
  hblk = (width - _L) // nheads if nheads > 1 else width
  pltpu.sync_copy(zeros_hbm, acc.at[pl.ds(sid * _ROWS_PER_TILE,
                                          _ROWS_PER_TILE)])
  plsc.subcore_barrier()

  iota = lax.iota(jnp.int32, _L)
  ngrp = chunk // _L

  if split_cores:
    base = (cid * _NS + sid) * per_tile_edges
  else:
    base = sid * per_tile_edges
  end = base + per_tile_edges

  def clamp_off(k):
    return jnp.minimum(base + k * chunk, _E - chunk)

  def fire_idx(k, b):
    off = clamp_off(k)
    pltpu.async_copy(ei_hbm.at[pl.ds(2 * off, 2 * chunk)], idxraw_v[b],
                     sem_i[b])

  def build(k, b):
    pltpu.make_async_copy(ei_hbm.at[pl.ds(0, 2 * chunk)], idxraw_v[b],
                          sem_i[b]).wait()
    off_raw = base + k * chunk
    off = clamp_off(k)
    for g in range(ngrp):
      lanes = g * _L + iota
      sv = plsc.load_gather(idxraw_v[b], [2 * lanes])
      tv = plsc.load_gather(idxraw_v[b], [2 * lanes + 1])
      if not split_cores:
        sv = sv + cid * _N
      gidx = off + lanes
      valid = (gidx >= off_raw) & (gidx < end)
      tv = jnp.where(valid, tv, _N + 64)
      src_v[b][pl.ds(g * _L, _L)] = sv
      trg_v[b][pl.ds(g * _L, _L)] = tv

  def fire_gather(b):
    pltpu.async_copy(table_hbm.at[src_v[b]], rows_v[b], sem_g[b])
    pltpu.async_copy(atrg_hbm.at[trg_v[b]], atrg_v[b], sem_g[b])

  def wait_gather(b):
    pltpu.make_async_copy(table_hbm.at[src_v[b]], rows_v[b], sem_g[b]).wait()
    pltpu.make_async_copy(atrg_hbm.at[trg_v[b]], atrg_v[b], sem_g[b]).wait()

  def wait_scatter(b):
    pltpu.make_async_copy(staged_v[b], acc.at[strg_v[b]], sem_s[b]).wait()

  def compute_scale(b):
    for g in range(ngrp):
      ridx = g * _L + iota
      for h in range(nheads):
        if nheads > 1:
          acol = jnp.full((_L,), 128 + h, jnp.int32)
          tcol = jnp.full((_L,), 2 * cid + h, jnp.int32)
        else:
          acol = jnp.full((_L,), 3, jnp.int32)
          tcol = jnp.full((_L,), 0, jnp.int32)
        a = plsc.load_gather(rows_v[b], [ridx, acol])
        t = plsc.load_gather(atrg_v[b], [ridx, tcol])
        e = a + t
        e = jnp.where(e >= 0.0, e, 0.2 * e)
        wbuf_v[pl.ds(h * chunk + g * _L, _L)] = jnp.exp(e)
      strg_v[b][pl.ds(g * _L, _L)] = trg_v[b][pl.ds(g * _L, _L)]
    for j in range(chunk):
      wb = [plsc.load_gather(wbuf_v, [jnp.full((_L,), h * chunk + j,
                                               jnp.int32)])
            for h in range(nheads)]
      if nheads > 1:
        for c in range(width // _L - 1):
          h = (c * _L) // hblk
          staged_v[b][j, pl.ds(c * _L, _L)] = (
              rows_v[b][j, pl.ds(c * _L, _L)] * wb[h])
        tail = jnp.where(iota == 0, wb[0],
                         jnp.where(iota == 1, wb[1], 0.0))
        staged_v[b][j, pl.ds(width - _L, _L)] = tail
      else:
        staged_v[b][j, pl.ds(0, _L)] = rows_v[b][j, pl.ds(0, _L)] * wb[0]

  fire_idx(jnp.int32(0), 0)
  fire_idx(jnp.int32(1), 1)
  build(jnp.int32(0), 0)
  fire_gather(0)

  def pair_body(p, carry):
    for b in (0, 1):
      k = 2 * p + b
      build(k + 1, 1 - b)
      fire_gather(1 - b)
      fire_idx(k + 2, b)
      @pl.when(p > 0)
      def _():
        wait_scatter(b)
      wait_gather(b)
      compute_scale(b)
      pltpu.async_copy(staged_v[b], acc.at[strg_v[b]], sem_s[b], add=True)
    return carry

  lax.fori_loop(0, npairs, pair_body, 0)
  wait_scatter(0)
  wait_scatter(1)
  wait_gather(0)
  pltpu.make_async_copy(ei_hbm.at[pl.ds(0, 2 * chunk)], idxraw_v[1],
                        sem_i[1]).wait()
  plsc.subcore_barrier()
  r0 = sid * _ROWS_PER_TILE
  pltpu.sync_copy(acc.at[pl.ds(r0, _ROWS_PER_TILE)],
                  out_hbm.at[cid, pl.ds(r0, _ROWS_PER_TILE)])


def _sc_edge_call(width, nheads, per_tile_edges, split_cores, chunk,
                  table, atrg_t, ei, zeros):
  mesh = plsc.VectorSubcoreMesh(core_axis_name="c", subcore_axis_name="s",
                                num_cores=_NC, num_subcores=_NS)
  body = functools.partial(_edge_kernel, width, nheads, per_tile_edges,
                           split_cores, chunk)
  fn = pl.kernel(
      body,
      out_type=jax.ShapeDtypeStruct((_NC, _NPAD, width), jnp.float32),
      mesh=mesh,
      compiler_params=pltpu.CompilerParams(use_tc_tiling_on_sc=False,
                                           needs_layout_passes=False),
      scratch_types=[
          [pltpu.VMEM((2 * chunk,), jnp.int32) for _ in range(2)],
          [pltpu.VMEM((chunk,), jnp.int32) for _ in range(2)],
          [pltpu.VMEM((chunk,), jnp.int32) for _ in range(2)],
          [pltpu.VMEM((chunk,), jnp.int32) for _ in range(2)],
          [pltpu.VMEM((chunk, width), jnp.float32) for _ in range(2)],
          [pltpu.VMEM((chunk, 16), jnp.float32) for _ in range(2)],
          [pltpu.VMEM((chunk, width), jnp.float32) for _ in range(2)],
          pltpu.VMEM((nheads * chunk,), jnp.float32),
          pltpu.VMEM_SHARED((_NPAD, width), jnp.float32),
          [pltpu.SemaphoreType.DMA for _ in range(2)],
          [pltpu.SemaphoreType.DMA for _ in range(2)],
          [pltpu.SemaphoreType.DMA for _ in range(2)],
      ],
  )
  return fn(table, atrg_t, ei, zeros)


def _k3_body(acc, w2r, as2, at2, table2_ref, atrg2_ref):
  a = acc[...]
  bn = a.shape[1]
  eps = 1e-16
  parts = []
  for c in range(2):
    num = a[c, :, 0:128]
    d0 = jnp.broadcast_to(a[c, :, 128:129], (bn, 64))
    d1 = jnp.broadcast_to(a[c, :, 129:130], (bn, 64))
    den = jnp.concatenate([d0, d1], axis=1)
    parts.append(num / (den + eps))
  out1 = jnp.concatenate(parts, axis=1)
  emb2 = jnp.where(out1 > 0.0, out1, jnp.exp(out1) - 1.0)
  hp2 = jnp.dot(emb2, w2r[...], preferred_element_type=jnp.float32)
  asrc2 = jnp.dot(hp2, as2[...], preferred_element_type=jnp.float32)
  atrg2 = jnp.dot(hp2, at2[...], preferred_element_type=jnp.float32)
  ones = jnp.ones((bn, 1), jnp.float32)
  table2_ref[...] = jnp.concatenate(
      [hp2, ones, asrc2, jnp.zeros((bn, 12), jnp.float32)], axis=1)
  atrg2_ref[...] = jnp.concatenate(
      [atrg2, jnp.zeros((bn, 15), jnp.float32)], axis=1)


def _k3(acc1, w2r, as2, at2):
  bn = 1000
  grid = _N // bn
  return pl.pallas_call(
      _k3_body,
      grid=(grid,),
      in_specs=[
          pl.BlockSpec((2, bn, _W1), lambda i: (0, i, 0)),
          pl.BlockSpec((256, 2), lambda i: (0, 0)),
          pl.BlockSpec((2, 1), lambda i: (0, 0)),
          pl.BlockSpec((2, 1), lambda i: (0, 0)),
      ],
      out_specs=[
          pl.BlockSpec((bn, 16), lambda i: (i, 0)),
          pl.BlockSpec((bn, 16), lambda i: (i, 0)),
      ],
      out_shape=[
          jax.ShapeDtypeStruct((_N, 16), jnp.float32),
          jax.ShapeDtypeStruct((_N, 16), jnp.float32),
      ],
  )(acc1, w2r, as2, at2)


def _k5_body(acc, out_ref):
  a = acc[...]
  num = a[0, :, 0:2] + a[1, :, 0:2]
  den = a[0, :, 2:3] + a[1, :, 2:3]
  x = num / (den + 1e-16)
  m = jnp.max(x, axis=1, keepdims=True)
  lse = m + jnp.log(jnp.sum(jnp.exp(x - m), axis=1, keepdims=True))
  out_ref[...] = x - lse


def _k5(acc2):
  return pl.pallas_call(
      _k5_body,
      out_shape=jax.ShapeDtypeStruct((_N, 2), jnp.float32),
  )(acc2)


@jax.jit
def kernel(static_emb, dynamic_emb_0, dynamic_emb_1, w1, attn_src1,
           attn_trg1, w2, attn_src2, attn_trg2, edge_index):
  w2d = jnp.transpose(w1, (1, 0, 2)).reshape(128, 256)
  bsrc = jax.scipy.linalg.block_diag(*[attn_src1[h] for h in range(4)])
  btrg = jax.scipy.linalg.block_diag(*[attn_trg1[h] for h in range(4)])
  w2r = w2[0]
  as2 = attn_src2[0]
  at2 = attn_trg2[0]

  table1, atrg1t = _k1(dynamic_emb_0, dynamic_emb_1, static_emb,
                       w2d, bsrc, btrg)
  table1 = table1.reshape(2 * _N, _W1)

  ei = edge_index.T.reshape(-1)
  z1 = jnp.zeros((_ROWS_PER_TILE, _W1), jnp.float32)
  acc1 = _sc_edge_call(_W1, 2, _E // _NS, False, 48,
                       table1, atrg1t, ei, z1)[:, :_N]

  table2, atrg2t = _k3(acc1, w2r, as2, at2)

  z2 = jnp.zeros((_ROWS_PER_TILE, _W2), jnp.float32)
  acc2 = _sc_edge_call(_W2, 1, _E // (_NC * _NS), True, 160,
                       table2, atrg2t, ei, z2)[:, :_N]

  return _k5(acc2)
